# baseline (speedup 1.0000x reference)
import functools

import jax
import jax.numpy as jnp
from jax import lax
from jax.experimental import pallas as pl
from jax.experimental.pallas import tpu as pltpu
from jax.experimental.pallas import tpu_sc as plsc

N_NODES = 10000
N_EDGES = 320000
D = 128
NC, NS, L = 2, 16, 16
NW = NC * NS
E0 = 96000
E1 = N_EDGES - E0
K = 40
SEG = 25
STRIPE = 640
TAIL = N_NODES - 15 * STRIPE
EBLK = 3200


def _mm_bias_body(x_ref, w_ref, b_ref, o_ref):
    o_ref[...] = (
        jnp.dot(x_ref[...], w_ref[...], preferred_element_type=jnp.float32)
        + b_ref[...]
    )


def _mm_bias(x, w, b, block_rows):
    m, k = x.shape
    n = w.shape[1]
    return pl.pallas_call(
        _mm_bias_body,
        grid=(m // block_rows,),
        in_specs=[
            pl.BlockSpec((block_rows, k), lambda i: (i, 0)),
            pl.BlockSpec((k, n), lambda i: (0, 0)),
            pl.BlockSpec((1, n), lambda i: (0, 0)),
        ],
        out_specs=pl.BlockSpec((block_rows, n), lambda i: (i, 0)),
        out_shape=jax.ShapeDtypeStruct((m, n), jnp.float32),
    )(x, w, b.reshape(1, n))


def _edge_mm_body(xt_ref, w_ref, b_ref, o_ref):
    y = jax.lax.dot_general(
        xt_ref[...], w_ref[...], (((0,), (0,)), ((), ())),
        preferred_element_type=jnp.float32,
    ) + b_ref[...]
    o_ref[...] = y


def _edge_mm_part(xt, w, b, eoff, ne):
    k = xt.shape[0]
    n = w.shape[1]
    off = eoff // EBLK
    return pl.pallas_call(
        _edge_mm_body,
        grid=(ne // EBLK,),
        in_specs=[
            pl.BlockSpec((k, EBLK), lambda i: (0, i + off)),
            pl.BlockSpec((k, n), lambda i: (0, 0)),
            pl.BlockSpec((1, n), lambda i: (0, 0)),
        ],
        out_specs=pl.BlockSpec((EBLK, n), lambda i: (i, 0)),
        out_shape=jax.ShapeDtypeStruct((ne, n), jnp.float32),
    )(xt, w, b.reshape(1, n))


def _final_body(p0_ref, p1_ref, w_ref, b_ref, o_ref):
    a = p0_ref[0] + p0_ref[1] + p1_ref[0] + p1_ref[1]
    o_ref[...] = (
        jnp.dot(a, w_ref[...], preferred_element_type=jnp.float32) + b_ref[...]
    )


def _final_mm(p0, p1, w, b, block_rows):
    _, m, n = p0.shape
    pspec = pl.BlockSpec((2, block_rows, n), lambda i: (0, i, 0))
    return pl.pallas_call(
        _final_body,
        grid=(m // block_rows,),
        in_specs=[
            pspec,
            pspec,
            pl.BlockSpec((n, n), lambda i: (0, 0)),
            pl.BlockSpec((1, n), lambda i: (0, 0)),
        ],
        out_specs=pl.BlockSpec((block_rows, n), lambda i: (i, 0)),
        out_shape=jax.ShapeDtypeStruct((m, n), jnp.float32),
    )(p0, p1, w, b.reshape(1, n))


_mesh = plsc.VectorSubcoreMesh(core_axis_name="c", subcore_axis_name="s")


def _make_sc_half(half, epw, nseg):
    @functools.partial(
        pl.kernel,
        out_type=jax.ShapeDtypeStruct((NC, N_NODES, D), jnp.float32),
        mesh=_mesh,
        scratch_types=[
            pltpu.VMEM((SEG, K), jnp.int32),
            pltpu.VMEM((SEG, K), jnp.int32),
            pltpu.VMEM((K, D), jnp.float32),
            pltpu.VMEM((K, D), jnp.float32),
            pltpu.VMEM((K, D), jnp.float32),
            pltpu.VMEM((K, D), jnp.float32),
            pltpu.VMEM((K, D), jnp.float32),
            pltpu.VMEM((K, D), jnp.float32),
            pltpu.VMEM_SHARED((N_NODES, D), jnp.float32),
            pltpu.SemaphoreType.DMA,
            pltpu.SemaphoreType.DMA,
            pltpu.SemaphoreType.DMA,
            pltpu.SemaphoreType.DMA,
            pltpu.SemaphoreType.DMA,
            pltpu.SemaphoreType.DMA,
        ],
        name=f"sc_gather_mul_scatter_h{half}",
    )
    def _sc_half(
        node_t, edge_t, src5, dst5, out,
        src_v, dst_v, gat0, gat1, edg0, edg1, sb0, sb1, agg_sh,
        gsem0, gsem1, esem0, esem1, ssem0, ssem1,
    ):
        c = lax.axis_index("c")
        s = lax.axis_index("s")
        wid = s * NC + c
        gat = (gat0, gat1)
        edg = (edg0, edg1)
        sb = (sb0, sb1)
        gsem = (gsem0, gsem1)
        esem = (esem0, esem1)
        ssem = (ssem0, ssem1)

        zvec = jnp.zeros((L,), jnp.float32)

        def zrow(r, carry):
            for v in range(D // L):
                gat0[r, pl.ds(v * L, L)] = zvec
            return carry

        lax.fori_loop(0, K, zrow, 0)
        base = s * STRIPE
        for z in range(TAIL // K):
            pltpu.sync_copy(gat0, agg_sh.at[pl.ds(base + z * K, K)])

        @pl.when(s < NS - 1)
        def _zero_rest():
            for z in range(TAIL // K, STRIPE // K):
                pltpu.sync_copy(gat0, agg_sh.at[pl.ds(base + z * K, K)])

        plsc.subcore_barrier()

        ebase = wid * epw

        def seg_body(seg, carry):
            soff = seg * SEG
            pltpu.sync_copy(src5.at[wid, seg], src_v)
            pltpu.sync_copy(dst5.at[wid, seg], dst_v)

            def fetch(l, b):
                pltpu.async_copy(
                    edge_t.at[pl.ds(ebase + (soff + l) * K, K)],
                    edg[b], esem[b])
                pltpu.async_copy(node_t.at[src_v.at[l]], gat[b], gsem[b])

            def wait_fetch(l, b):
                pltpu.make_async_copy(
                    edge_t.at[pl.ds(ebase + (soff + l) * K, K)],
                    edg[b], esem[b]).wait()
                pltpu.make_async_copy(
                    node_t.at[src_v.at[l]], gat[b], gsem[b]).wait()

            def multiply(b):
                ga, eb, sbb = gat[b], edg[b], sb[b]

                def mul(e, inner):
                    for v in range(D // L):
                        sl = pl.ds(v * L, L)
                        sbb[e, sl] = ga[e, sl] * eb[e, sl]
                    return inner

                lax.fori_loop(0, K, mul, 0)

            def scatter(l, b):
                pltpu.async_copy(
                    sb[b], agg_sh.at[dst_v.at[l]], ssem[b], add=True)

            def wait_scatter(l, b):
                pltpu.make_async_copy(
                    sb[b], agg_sh.at[dst_v.at[l]], ssem[b]).wait()

            fetch(0, 0)

            def pair(p, inner):
                la, lb = 2 * p, 2 * p + 1
                fetch(lb, 1)
                wait_fetch(la, 0)

                @pl.when(p > 0)
                def _ws0():
                    wait_scatter(la - 2, 0)

                multiply(0)
                scatter(la, 0)
                fetch(la + 2, 0)
                wait_fetch(lb, 1)

                @pl.when(p > 0)
                def _ws1():
                    wait_scatter(lb - 2, 1)

                multiply(1)
                scatter(lb, 1)
                return inner

            lax.fori_loop(0, SEG // 2, pair, 0)

            wait_fetch(SEG - 1, 0)
            wait_scatter(SEG - 3, 0)
            multiply(0)
            scatter(SEG - 1, 0)
            wait_scatter(SEG - 1, 0)
            wait_scatter(SEG - 2, 1)
            return carry

        lax.fori_loop(0, nseg, seg_body, 0)

        plsc.subcore_barrier()
        pltpu.sync_copy(
            agg_sh.at[pl.ds(base, TAIL)],
            out.at[c, pl.ds(base, TAIL)],
        )

        @pl.when(s < NS - 1)
        def _write_rest():
            pltpu.sync_copy(
                agg_sh.at[pl.ds(base + TAIL, STRIPE - TAIL)],
                out.at[c, pl.ds(base + TAIL, STRIPE - TAIL)],
            )

    return _sc_half


_sc_half0 = _make_sc_half(0, E0 // NW, E0 // NW // K // SEG)
_sc_half1 = _make_sc_half(1, E1 // NW, E1 // NW // K // SEG)


def kernel(node_features, edge_features, edge_indices,
           W_node, b_node, W_edge, b_edge, W_out, b_out):
    node_t = _mm_bias(node_features, W_node, b_node, 1000)
    eft = edge_features.T
    eit = edge_indices.astype(jnp.int32).T
    src0 = eit[0, :E0].reshape(NW, E0 // NW // K // SEG, SEG, K)
    dst0 = eit[1, :E0].reshape(NW, E0 // NW // K // SEG, SEG, K)
    src1 = eit[0, E0:].reshape(NW, E1 // NW // K // SEG, SEG, K)
    dst1 = eit[1, E0:].reshape(NW, E1 // NW // K // SEG, SEG, K)
    edge_t0 = _edge_mm_part(eft, W_edge, b_edge, 0, E0)
    p0 = _sc_half0(node_t, edge_t0, src0, dst0)
    edge_t1 = _edge_mm_part(eft, W_edge, b_edge, E0, E1)
    p1 = _sc_half1(node_t, edge_t1, src1, dst1)
    return _final_mm(p0, p1, W_out, b_out, 1000)


# speedup vs baseline: 1.0273x; 1.0273x over previous
import functools

import jax
import jax.numpy as jnp
from jax import lax
from jax.experimental import pallas as pl
from jax.experimental.pallas import tpu as pltpu
from jax.experimental.pallas import tpu_sc as plsc

N_NODES = 10000
N_EDGES = 320000
D = 128
NC, NS, L = 2, 16, 16
NW = NC * NS
E0 = 128000
E1 = N_EDGES - E0
K = 40
SEG = 25
STRIPE = 640
TAIL = N_NODES - 15 * STRIPE
EBLK = 3200


def _mm_bias_body(x_ref, w_ref, b_ref, o_ref):
    o_ref[...] = (
        jnp.dot(x_ref[...], w_ref[...], preferred_element_type=jnp.float32)
        + b_ref[...]
    )


def _mm_bias(x, w, b, block_rows):
    m, k = x.shape
    n = w.shape[1]
    return pl.pallas_call(
        _mm_bias_body,
        grid=(m // block_rows,),
        in_specs=[
            pl.BlockSpec((block_rows, k), lambda i: (i, 0)),
            pl.BlockSpec((k, n), lambda i: (0, 0)),
            pl.BlockSpec((1, n), lambda i: (0, 0)),
        ],
        out_specs=pl.BlockSpec((block_rows, n), lambda i: (i, 0)),
        out_shape=jax.ShapeDtypeStruct((m, n), jnp.float32),
    )(x, w, b.reshape(1, n))


def _edge_mm_body(xt_ref, w_ref, b_ref, o_ref):
    y = jax.lax.dot_general(
        xt_ref[...], w_ref[...], (((0,), (0,)), ((), ())),
        preferred_element_type=jnp.float32,
    ) + b_ref[...]
    o_ref[...] = y


def _edge_mm_part(xt, w, b, eoff, ne):
    k = xt.shape[0]
    n = w.shape[1]
    off = eoff // EBLK
    return pl.pallas_call(
        _edge_mm_body,
        grid=(ne // EBLK,),
        in_specs=[
            pl.BlockSpec((k, EBLK), lambda i: (0, i + off)),
            pl.BlockSpec((k, n), lambda i: (0, 0)),
            pl.BlockSpec((1, n), lambda i: (0, 0)),
        ],
        out_specs=pl.BlockSpec((EBLK, n), lambda i: (i, 0)),
        out_shape=jax.ShapeDtypeStruct((ne, n), jnp.float32),
    )(xt, w, b.reshape(1, n))


def _final_body(p0_ref, p1_ref, w_ref, b_ref, o_ref):
    a = p0_ref[0] + p0_ref[1] + p1_ref[0] + p1_ref[1]
    o_ref[...] = (
        jnp.dot(a, w_ref[...], preferred_element_type=jnp.float32) + b_ref[...]
    )


def _final_mm(p0, p1, w, b, block_rows):
    _, m, n = p0.shape
    pspec = pl.BlockSpec((2, block_rows, n), lambda i: (0, i, 0))
    return pl.pallas_call(
        _final_body,
        grid=(m // block_rows,),
        in_specs=[
            pspec,
            pspec,
            pl.BlockSpec((n, n), lambda i: (0, 0)),
            pl.BlockSpec((1, n), lambda i: (0, 0)),
        ],
        out_specs=pl.BlockSpec((block_rows, n), lambda i: (i, 0)),
        out_shape=jax.ShapeDtypeStruct((m, n), jnp.float32),
    )(p0, p1, w, b.reshape(1, n))


_mesh = plsc.VectorSubcoreMesh(core_axis_name="c", subcore_axis_name="s")


def _make_sc_half(half, epw, nseg):
    @functools.partial(
        pl.kernel,
        out_type=jax.ShapeDtypeStruct((NC, N_NODES, D), jnp.float32),
        mesh=_mesh,
        scratch_types=[
            pltpu.VMEM((SEG, K), jnp.int32),
            pltpu.VMEM((SEG, K), jnp.int32),
            pltpu.VMEM((K, D), jnp.float32),
            pltpu.VMEM((K, D), jnp.float32),
            pltpu.VMEM((K, D), jnp.float32),
            pltpu.VMEM((K, D), jnp.float32),
            pltpu.VMEM((K, D), jnp.float32),
            pltpu.VMEM((K, D), jnp.float32),
            pltpu.VMEM_SHARED((N_NODES, D), jnp.float32),
            pltpu.SemaphoreType.DMA,
            pltpu.SemaphoreType.DMA,
            pltpu.SemaphoreType.DMA,
            pltpu.SemaphoreType.DMA,
            pltpu.SemaphoreType.DMA,
            pltpu.SemaphoreType.DMA,
        ],
        name=f"sc_gather_mul_scatter_h{half}",
    )
    def _sc_half(
        node_t, edge_t, src5, dst5, out,
        src_v, dst_v, gat0, gat1, edg0, edg1, sb0, sb1, agg_sh,
        gsem0, gsem1, esem0, esem1, ssem0, ssem1,
    ):
        c = lax.axis_index("c")
        s = lax.axis_index("s")
        wid = s * NC + c
        gat = (gat0, gat1)
        edg = (edg0, edg1)
        sb = (sb0, sb1)
        gsem = (gsem0, gsem1)
        esem = (esem0, esem1)
        ssem = (ssem0, ssem1)

        zvec = jnp.zeros((L,), jnp.float32)

        def zrow(r, carry):
            for v in range(D // L):
                gat0[r, pl.ds(v * L, L)] = zvec
            return carry

        lax.fori_loop(0, K, zrow, 0)
        base = s * STRIPE
        for z in range(TAIL // K):
            pltpu.sync_copy(gat0, agg_sh.at[pl.ds(base + z * K, K)])

        @pl.when(s < NS - 1)
        def _zero_rest():
            for z in range(TAIL // K, STRIPE // K):
                pltpu.sync_copy(gat0, agg_sh.at[pl.ds(base + z * K, K)])

        plsc.subcore_barrier()

        ebase = wid * epw

        def seg_body(seg, carry):
            soff = seg * SEG
            pltpu.sync_copy(src5.at[wid, seg], src_v)
            pltpu.sync_copy(dst5.at[wid, seg], dst_v)

            def fetch(l, b):
                pltpu.async_copy(
                    edge_t.at[pl.ds(ebase + (soff + l) * K, K)],
                    edg[b], esem[b])
                pltpu.async_copy(node_t.at[src_v.at[l]], gat[b], gsem[b])

            def wait_fetch(l, b):
                pltpu.make_async_copy(
                    edge_t.at[pl.ds(ebase + (soff + l) * K, K)],
                    edg[b], esem[b]).wait()
                pltpu.make_async_copy(
                    node_t.at[src_v.at[l]], gat[b], gsem[b]).wait()

            def multiply(b):
                ga, eb, sbb = gat[b], edg[b], sb[b]

                def mul(e, inner):
                    for v in range(D // L):
                        sl = pl.ds(v * L, L)
                        sbb[e, sl] = ga[e, sl] * eb[e, sl]
                    return inner

                lax.fori_loop(0, K, mul, 0)

            def scatter(l, b):
                pltpu.async_copy(
                    sb[b], agg_sh.at[dst_v.at[l]], ssem[b], add=True)

            def wait_scatter(l, b):
                pltpu.make_async_copy(
                    sb[b], agg_sh.at[dst_v.at[l]], ssem[b]).wait()

            fetch(0, 0)

            def pair(p, inner):
                la, lb = 2 * p, 2 * p + 1
                fetch(lb, 1)
                wait_fetch(la, 0)

                @pl.when(p > 0)
                def _ws0():
                    wait_scatter(la - 2, 0)

                multiply(0)
                scatter(la, 0)
                fetch(la + 2, 0)
                wait_fetch(lb, 1)

                @pl.when(p > 0)
                def _ws1():
                    wait_scatter(lb - 2, 1)

                multiply(1)
                scatter(lb, 1)
                return inner

            lax.fori_loop(0, SEG // 2, pair, 0)

            wait_fetch(SEG - 1, 0)
            wait_scatter(SEG - 3, 0)
            multiply(0)
            scatter(SEG - 1, 0)
            wait_scatter(SEG - 1, 0)
            wait_scatter(SEG - 2, 1)
            return carry

        lax.fori_loop(0, nseg, seg_body, 0)

        plsc.subcore_barrier()
        pltpu.sync_copy(
            agg_sh.at[pl.ds(base, TAIL)],
            out.at[c, pl.ds(base, TAIL)],
        )

        @pl.when(s < NS - 1)
        def _write_rest():
            pltpu.sync_copy(
                agg_sh.at[pl.ds(base + TAIL, STRIPE - TAIL)],
                out.at[c, pl.ds(base + TAIL, STRIPE - TAIL)],
            )

    return _sc_half


_sc_half0 = _make_sc_half(0, E0 // NW, E0 // NW // K // SEG)
_sc_half1 = _make_sc_half(1, E1 // NW, E1 // NW // K // SEG)


def kernel(node_features, edge_features, edge_indices,
           W_node, b_node, W_edge, b_edge, W_out, b_out):
    node_t = _mm_bias(node_features, W_node, b_node, 1000)
    eft = edge_features.T
    eit = edge_indices.astype(jnp.int32).T
    src0 = eit[0, :E0].reshape(NW, E0 // NW // K // SEG, SEG, K)
    dst0 = eit[1, :E0].reshape(NW, E0 // NW // K // SEG, SEG, K)
    src1 = eit[0, E0:].reshape(NW, E1 // NW // K // SEG, SEG, K)
    dst1 = eit[1, E0:].reshape(NW, E1 // NW // K // SEG, SEG, K)
    edge_t0 = _edge_mm_part(eft, W_edge, b_edge, 0, E0)
    p0 = _sc_half0(node_t, edge_t0, src0, dst0)
    edge_t1 = _edge_mm_part(eft, W_edge, b_edge, E0, E1)
    p1 = _sc_half1(node_t, edge_t1, src1, dst1)
    return _final_mm(p0, p1, W_out, b_out, 1000)


# speedup vs baseline: 1.0508x; 1.0229x over previous
import functools

import jax
import jax.numpy as jnp
from jax import lax
from jax.experimental import pallas as pl
from jax.experimental.pallas import tpu as pltpu
from jax.experimental.pallas import tpu_sc as plsc

N_NODES = 10000
N_EDGES = 320000
D = 128
NC, NS, L = 2, 16, 16
NW = NC * NS
E0 = 160000
E1 = N_EDGES - E0
K = 40
SEG = 25
STRIPE = 640
TAIL = N_NODES - 15 * STRIPE
EBLK = 3200


def _mm_bias_body(x_ref, w_ref, b_ref, o_ref):
    o_ref[...] = (
        jnp.dot(x_ref[...], w_ref[...], preferred_element_type=jnp.float32)
        + b_ref[...]
    )


def _mm_bias(x, w, b, block_rows):
    m, k = x.shape
    n = w.shape[1]
    return pl.pallas_call(
        _mm_bias_body,
        grid=(m // block_rows,),
        in_specs=[
            pl.BlockSpec((block_rows, k), lambda i: (i, 0)),
            pl.BlockSpec((k, n), lambda i: (0, 0)),
            pl.BlockSpec((1, n), lambda i: (0, 0)),
        ],
        out_specs=pl.BlockSpec((block_rows, n), lambda i: (i, 0)),
        out_shape=jax.ShapeDtypeStruct((m, n), jnp.float32),
    )(x, w, b.reshape(1, n))


def _edge_mm_body(xt_ref, w_ref, b_ref, o_ref):
    y = jax.lax.dot_general(
        xt_ref[...], w_ref[...], (((0,), (0,)), ((), ())),
        preferred_element_type=jnp.float32,
    ) + b_ref[...]
    o_ref[...] = y


def _edge_mm_part(xt, w, b, eoff, ne):
    k = xt.shape[0]
    n = w.shape[1]
    off = eoff // EBLK
    return pl.pallas_call(
        _edge_mm_body,
        grid=(ne // EBLK,),
        in_specs=[
            pl.BlockSpec((k, EBLK), lambda i: (0, i + off)),
            pl.BlockSpec((k, n), lambda i: (0, 0)),
            pl.BlockSpec((1, n), lambda i: (0, 0)),
        ],
        out_specs=pl.BlockSpec((EBLK, n), lambda i: (i, 0)),
        out_shape=jax.ShapeDtypeStruct((ne, n), jnp.float32),
    )(xt, w, b.reshape(1, n))


def _final_body(p0_ref, p1_ref, w_ref, b_ref, o_ref):
    a = p0_ref[0] + p0_ref[1] + p1_ref[0] + p1_ref[1]
    o_ref[...] = (
        jnp.dot(a, w_ref[...], preferred_element_type=jnp.float32) + b_ref[...]
    )


def _final_mm(p0, p1, w, b, block_rows):
    _, m, n = p0.shape
    pspec = pl.BlockSpec((2, block_rows, n), lambda i: (0, i, 0))
    return pl.pallas_call(
        _final_body,
        grid=(m // block_rows,),
        in_specs=[
            pspec,
            pspec,
            pl.BlockSpec((n, n), lambda i: (0, 0)),
            pl.BlockSpec((1, n), lambda i: (0, 0)),
        ],
        out_specs=pl.BlockSpec((block_rows, n), lambda i: (i, 0)),
        out_shape=jax.ShapeDtypeStruct((m, n), jnp.float32),
    )(p0, p1, w, b.reshape(1, n))


_mesh = plsc.VectorSubcoreMesh(core_axis_name="c", subcore_axis_name="s")


def _make_sc_half(half, epw, nseg):
    @functools.partial(
        pl.kernel,
        out_type=jax.ShapeDtypeStruct((NC, N_NODES, D), jnp.float32),
        mesh=_mesh,
        scratch_types=[
            pltpu.VMEM((SEG, K), jnp.int32),
            pltpu.VMEM((SEG, K), jnp.int32),
            pltpu.VMEM((K, D), jnp.float32),
            pltpu.VMEM((K, D), jnp.float32),
            pltpu.VMEM((K, D), jnp.float32),
            pltpu.VMEM((K, D), jnp.float32),
            pltpu.VMEM((K, D), jnp.float32),
            pltpu.VMEM((K, D), jnp.float32),
            pltpu.VMEM_SHARED((N_NODES, D), jnp.float32),
            pltpu.SemaphoreType.DMA,
            pltpu.SemaphoreType.DMA,
            pltpu.SemaphoreType.DMA,
            pltpu.SemaphoreType.DMA,
            pltpu.SemaphoreType.DMA,
            pltpu.SemaphoreType.DMA,
        ],
        name=f"sc_gather_mul_scatter_h{half}",
    )
    def _sc_half(
        node_t, edge_t, src5, dst5, out,
        src_v, dst_v, gat0, gat1, edg0, edg1, sb0, sb1, agg_sh,
        gsem0, gsem1, esem0, esem1, ssem0, ssem1,
    ):
        c = lax.axis_index("c")
        s = lax.axis_index("s")
        wid = s * NC + c
        gat = (gat0, gat1)
        edg = (edg0, edg1)
        sb = (sb0, sb1)
        gsem = (gsem0, gsem1)
        esem = (esem0, esem1)
        ssem = (ssem0, ssem1)

        zvec = jnp.zeros((L,), jnp.float32)

        def zrow(r, carry):
            for v in range(D // L):
                gat0[r, pl.ds(v * L, L)] = zvec
            return carry

        lax.fori_loop(0, K, zrow, 0)
        base = s * STRIPE
        for z in range(TAIL // K):
            pltpu.sync_copy(gat0, agg_sh.at[pl.ds(base + z * K, K)])

        @pl.when(s < NS - 1)
        def _zero_rest():
            for z in range(TAIL // K, STRIPE // K):
                pltpu.sync_copy(gat0, agg_sh.at[pl.ds(base + z * K, K)])

        plsc.subcore_barrier()

        ebase = wid * epw

        def seg_body(seg, carry):
            soff = seg * SEG
            pltpu.sync_copy(src5.at[wid, seg], src_v)
            pltpu.sync_copy(dst5.at[wid, seg], dst_v)

            def fetch(l, b):
                pltpu.async_copy(
                    edge_t.at[pl.ds(ebase + (soff + l) * K, K)],
                    edg[b], esem[b])
                pltpu.async_copy(node_t.at[src_v.at[l]], gat[b], gsem[b])

            def wait_fetch(l, b):
                pltpu.make_async_copy(
                    edge_t.at[pl.ds(ebase + (soff + l) * K, K)],
                    edg[b], esem[b]).wait()
                pltpu.make_async_copy(
                    node_t.at[src_v.at[l]], gat[b], gsem[b]).wait()

            def multiply(b):
                ga, eb, sbb = gat[b], edg[b], sb[b]

                def mul(e, inner):
                    for v in range(D // L):
                        sl = pl.ds(v * L, L)
                        sbb[e, sl] = ga[e, sl] * eb[e, sl]
                    return inner

                lax.fori_loop(0, K, mul, 0)

            def scatter(l, b):
                pltpu.async_copy(
                    sb[b], agg_sh.at[dst_v.at[l]], ssem[b], add=True)

            def wait_scatter(l, b):
                pltpu.make_async_copy(
                    sb[b], agg_sh.at[dst_v.at[l]], ssem[b]).wait()

            fetch(0, 0)

            def pair(p, inner):
                la, lb = 2 * p, 2 * p + 1
                fetch(lb, 1)
                wait_fetch(la, 0)

                @pl.when(p > 0)
                def _ws0():
                    wait_scatter(la - 2, 0)

                multiply(0)
                scatter(la, 0)
                fetch(la + 2, 0)
                wait_fetch(lb, 1)

                @pl.when(p > 0)
                def _ws1():
                    wait_scatter(lb - 2, 1)

                multiply(1)
                scatter(lb, 1)
                return inner

            lax.fori_loop(0, SEG // 2, pair, 0)

            wait_fetch(SEG - 1, 0)
            wait_scatter(SEG - 3, 0)
            multiply(0)
            scatter(SEG - 1, 0)
            wait_scatter(SEG - 1, 0)
            wait_scatter(SEG - 2, 1)
            return carry

        lax.fori_loop(0, nseg, seg_body, 0)

        plsc.subcore_barrier()
        pltpu.sync_copy(
            agg_sh.at[pl.ds(base, TAIL)],
            out.at[c, pl.ds(base, TAIL)],
        )

        @pl.when(s < NS - 1)
        def _write_rest():
            pltpu.sync_copy(
                agg_sh.at[pl.ds(base + TAIL, STRIPE - TAIL)],
                out.at[c, pl.ds(base + TAIL, STRIPE - TAIL)],
            )

    return _sc_half


_sc_half0 = _make_sc_half(0, E0 // NW, E0 // NW // K // SEG)
_sc_half1 = _make_sc_half(1, E1 // NW, E1 // NW // K // SEG)


def kernel(node_features, edge_features, edge_indices,
           W_node, b_node, W_edge, b_edge, W_out, b_out):
    node_t = _mm_bias(node_features, W_node, b_node, 1000)
    eft = edge_features.T
    eit = edge_indices.astype(jnp.int32).T
    src0 = eit[0, :E0].reshape(NW, E0 // NW // K // SEG, SEG, K)
    dst0 = eit[1, :E0].reshape(NW, E0 // NW // K // SEG, SEG, K)
    src1 = eit[0, E0:].reshape(NW, E1 // NW // K // SEG, SEG, K)
    dst1 = eit[1, E0:].reshape(NW, E1 // NW // K // SEG, SEG, K)
    edge_t0 = _edge_mm_part(eft, W_edge, b_edge, 0, E0)
    p0 = _sc_half0(node_t, edge_t0, src0, dst0)
    edge_t1 = _edge_mm_part(eft, W_edge, b_edge, E0, E1)
    p1 = _sc_half1(node_t, edge_t1, src1, dst1)
    return _final_mm(p0, p1, W_out, b_out, 1000)
